# baseline (speedup 1.0000x reference)
import jax
import jax.numpy as jnp
from jax import lax
from jax.experimental import pallas as pl
from jax.experimental.pallas import tpu as pltpu

K = 1024
E = 256
B = 8
T = 2048
TB = 2048


def _lane_fold(v):
    acc = v[:, 0:128]
    for o in range(128, v.shape[1], 128):
        acc = acc + v[:, o:o + 128]
    return acc


def _vq_kernel(x_ref, xp_ref, cb_ref, out_ref, idx_ref, smin_ref, sx_ref,
               sx2_ref, cbs_ref, cb2_ref, cbb_ref, dd_ref, mind_ref,
               x2_ref):
    s = pl.program_id(0)

    @pl.when(s == 0)
    def _():
        cb = cb_ref[...]
        cbs_ref[...] = -2.0 * cb
        cb2_ref[...] = jnp.sum(cb * cb, axis=1, keepdims=True)
        cbb_ref[...] = cb.astype(jnp.bfloat16)

    @pl.when(s >= 1)
    def _():
        dd = dd_ref[...]
        min_dd = mind_ref[...]
        iota = lax.broadcasted_iota(jnp.int32, dd.shape, 0)
        idx = jnp.min(jnp.where(dd == min_dd, iota, K), axis=0,
                      keepdims=True)
        onehot = (iota == idx).astype(jnp.bfloat16)
        g = lax.dot_general(cbb_ref[...], onehot, (((0,), (0,)), ((), ())),
                            preferred_element_type=jnp.float32)
        xp = xp_ref[0]
        out_ref[0] = xp + (g - xp)
        idx_ref[0] = idx
        x2 = x2_ref[...]
        smin_ref[0] = _lane_fold(min_dd + x2)
        sx_ref[0] = _lane_fold(jnp.sum(xp, axis=0, keepdims=True))
        sx2_ref[0] = _lane_fold(x2)

    @pl.when(s < B)
    def _():
        x = x_ref[0]
        m2 = lax.dot_general(cbs_ref[...], x, (((1,), (0,)), ((), ())),
                             preferred_element_type=jnp.float32)
        dd = m2 + cb2_ref[...]
        dd_ref[...] = dd
        mind_ref[...] = jnp.min(dd, axis=0, keepdims=True)
        x2_ref[...] = jnp.sum(x * x, axis=0, keepdims=True)


@jax.jit
def kernel(x, codebook):
    n_elem = B * E * T
    grid = (B + 1,)
    cur = lambda s: (jnp.minimum(s, B - 1), 0, 0)
    prev = lambda s: (jnp.maximum(s - 1, 0), 0, 0)
    out, idx, smin, sx, sx2 = pl.pallas_call(
        _vq_kernel,
        grid=grid,
        in_specs=[
            pl.BlockSpec((1, E, TB), cur),
            pl.BlockSpec((1, E, TB), prev),
            pl.BlockSpec((K, E), lambda s: (0, 0)),
        ],
        out_specs=[
            pl.BlockSpec((1, E, TB), prev),
            pl.BlockSpec((1, 1, TB), prev),
            pl.BlockSpec((1, 1, 128), prev),
            pl.BlockSpec((1, 1, 128), prev),
            pl.BlockSpec((1, 1, 128), prev),
        ],
        out_shape=[
            jax.ShapeDtypeStruct((B, E, T), jnp.float32),
            jax.ShapeDtypeStruct((B, 1, T), jnp.int32),
            jax.ShapeDtypeStruct((B, 1, 128), jnp.float32),
            jax.ShapeDtypeStruct((B, 1, 128), jnp.float32),
            jax.ShapeDtypeStruct((B, 1, 128), jnp.float32),
        ],
        scratch_shapes=[
            pltpu.VMEM((K, E), jnp.float32),
            pltpu.VMEM((K, 1), jnp.float32),
            pltpu.VMEM((K, E), jnp.bfloat16),
            pltpu.VMEM((K, TB), jnp.float32),
            pltpu.VMEM((1, TB), jnp.float32),
            pltpu.VMEM((1, TB), jnp.float32),
        ],
    )(x, x, codebook)

    sum_min = jnp.sum(smin)
    fit = sum_min / (B * T)
    commit_loss = sum_min / n_elem
    mean = jnp.sum(sx) / n_elem
    x_norm = jnp.sqrt(jnp.maximum(jnp.sum(sx2) / n_elem - mean * mean, 0.0))
    codebook_idxs = idx.reshape(B, T)
    return (out, commit_loss, fit, x_norm, codebook_idxs)


# speedup vs baseline: 1.1509x; 1.1509x over previous
import jax
import jax.numpy as jnp
from jax import lax
from jax.experimental import pallas as pl
from jax.experimental.pallas import tpu as pltpu

K = 1024
E = 256
B = 8
T = 2048
TB = 2048


def _lane_fold(v):
    acc = v[:, 0:128]
    for o in range(128, v.shape[1], 128):
        acc = acc + v[:, o:o + 128]
    return acc


def _vq_kernel(x_ref, cb_ref, out_ref, idx_ref, smin_ref, sx_ref, sx2_ref,
               cbs_ref, cb2_ref, cbb_ref):
    b = pl.program_id(0)

    @pl.when(b == 0)
    def _():
        cb = cb_ref[...]
        cbs_ref[...] = -2.0 * cb
        cb2_ref[...] = jnp.sum(cb * cb, axis=1, keepdims=True)
        cbb_ref[...] = cb.astype(jnp.bfloat16)

    x = x_ref[0]

    m2 = lax.dot_general(cbs_ref[...], x, (((1,), (0,)), ((), ())),
                         preferred_element_type=jnp.float32)
    dd = m2 + cb2_ref[...]

    min_dd = jnp.min(dd, axis=0, keepdims=True)
    idx = jnp.argmin(dd, axis=0).reshape(1, TB)
    iota = lax.broadcasted_iota(jnp.int32, dd.shape, 0)
    onehot = (iota == idx).astype(jnp.bfloat16)
    g = lax.dot_general(cbb_ref[...], onehot, (((0,), (0,)), ((), ())),
                        preferred_element_type=jnp.float32)

    out_ref[0] = x + (g - x)
    idx_ref[0] = idx

    x2 = jnp.sum(x * x, axis=0, keepdims=True)
    smin_ref[0] = _lane_fold(min_dd + x2)
    sx_ref[0] = _lane_fold(jnp.sum(x, axis=0, keepdims=True))
    sx2_ref[0] = _lane_fold(x2)


@jax.jit
def kernel(x, codebook):
    n_elem = B * E * T
    grid = (B,)
    out, idx, smin, sx, sx2 = pl.pallas_call(
        _vq_kernel,
        grid=grid,
        in_specs=[
            pl.BlockSpec((1, E, TB), lambda b: (b, 0, 0)),
            pl.BlockSpec((K, E), lambda b: (0, 0)),
        ],
        out_specs=[
            pl.BlockSpec((1, E, TB), lambda b: (b, 0, 0)),
            pl.BlockSpec((1, 1, TB), lambda b: (b, 0, 0)),
            pl.BlockSpec((1, 1, 128), lambda b: (b, 0, 0)),
            pl.BlockSpec((1, 1, 128), lambda b: (b, 0, 0)),
            pl.BlockSpec((1, 1, 128), lambda b: (b, 0, 0)),
        ],
        out_shape=[
            jax.ShapeDtypeStruct((B, E, T), jnp.float32),
            jax.ShapeDtypeStruct((B, 1, T), jnp.int32),
            jax.ShapeDtypeStruct((B, 1, 128), jnp.float32),
            jax.ShapeDtypeStruct((B, 1, 128), jnp.float32),
            jax.ShapeDtypeStruct((B, 1, 128), jnp.float32),
        ],
        scratch_shapes=[
            pltpu.VMEM((K, E), jnp.float32),
            pltpu.VMEM((K, 1), jnp.float32),
            pltpu.VMEM((K, E), jnp.bfloat16),
        ],
    )(x, codebook)

    sum_min = jnp.sum(smin)
    fit = sum_min / (B * T)
    commit_loss = sum_min / n_elem
    mean = jnp.sum(sx) / n_elem
    x_norm = jnp.sqrt(jnp.maximum(jnp.sum(sx2) / n_elem - mean * mean, 0.0))
    codebook_idxs = idx.reshape(B, T)
    return (out, commit_loss, fit, x_norm, codebook_idxs)
